# Initial kernel scaffold; baseline (speedup 1.0000x reference)
import jax
import jax.numpy as jnp
from jax.experimental import pallas as pl


def kernel(variable_prediction, graph_map, batch_variable_map, batch_function_map, edge_feature):
    raise NotImplementedError("write your pallas kernel here")


# speedup vs baseline: 129.4705x; 129.4705x over previous
import functools

import jax
import jax.numpy as jnp
from jax import lax
from jax.experimental import pallas as pl
from jax.experimental.pallas import tpu as pltpu
from jax.experimental.pallas import tpu_sc as plsc

NC = 2
NS = 16
L = 16
BATCH = 16


@functools.lru_cache(maxsize=None)
def _edge_pass(V, F, E):
    R = E // 128
    CH = 16
    NW = NC * NS
    full_chunks = R // CH
    tail_rows = R % CH
    per = full_chunks // NW
    rem = full_chunks % NW

    span = ((F + NS - 1) // NS + 7) // 8 * 8
    last = F - (NS - 1) * span
    z_full, z_tail = span // 2048, span % 2048
    zl_full, zl_tail = last // 2048, last % 2048

    mesh = plsc.VectorSubcoreMesh(core_axis_name="c", subcore_axis_name="s")

    @functools.partial(
        pl.kernel,
        out_type=(jax.ShapeDtypeStruct((F,), jnp.float32),
                  jax.ShapeDtypeStruct((F,), jnp.float32)),
        mesh=mesh,
        scratch_types=[
            pltpu.VMEM((V,), jnp.float32),
            pltpu.VMEM((CH, 128), jnp.int32),
            pltpu.VMEM((CH, 128), jnp.int32),
            pltpu.VMEM((CH, 128), jnp.float32),
            pltpu.VMEM((CH, 128), jnp.float32),
            pltpu.VMEM((2048,), jnp.float32),
            pltpu.VMEM_SHARED((F,), jnp.float32),
            pltpu.SemaphoreType.DMA,
        ],
        compiler_params=pltpu.CompilerParams(needs_layout_passes=False),
    )
    def k(vp_h, gm_h, ef_h, out0_h, out1_h, vp_v, var_v, fun_v, ef_v, ev_v,
          zb_v, acc_s, sem):
        cid = lax.axis_index("c")
        sid = lax.axis_index("s")
        wid = sid * NC + cid

        pltpu.sync_copy(vp_h, vp_v)

        def zb_body(i, _):
            zb_v[pl.ds(i * L, L)] = jnp.zeros((L,), jnp.float32)
            return 0
        lax.fori_loop(0, 2048 // L, zb_body, 0)

        base = sid * span

        def z_body(j, _):
            pltpu.sync_copy(zb_v, acc_s.at[pl.ds(base + j * 2048, 2048)])
            return 0
        nfull_z = jnp.where(sid == NS - 1, zl_full, z_full)
        lax.fori_loop(0, nfull_z, z_body, 0)
        if z_tail:
            @pl.when(sid != NS - 1)
            def _():
                pltpu.sync_copy(zb_v.at[pl.ds(0, z_tail)],
                                acc_s.at[pl.ds(base + z_full * 2048, z_tail)])
        if zl_tail:
            @pl.when(sid == NS - 1)
            def _():
                pltpu.sync_copy(zb_v.at[pl.ds(0, zl_tail)],
                                acc_s.at[pl.ds(base + zl_full * 2048, zl_tail)])

        plsc.subcore_barrier()

        nch = per + jnp.where(wid < rem, 1, 0)
        ch0 = wid * per + jnp.minimum(wid, rem)

        def compute_vectors(nvec_valid):
            def vec(i, _):
                r = i // 8
                c = i % 8
                idx = var_v[r, pl.ds(c * L, L)]
                vv = plsc.load_gather(vp_v, [idx])
                e = ef_v[r, pl.ds(c * L, L)]
                t = e * vv + (1.0 - e) * 0.5
                ev = jnp.where(t > 0.5, 1.0, 0.0)
                if nvec_valid is not None:
                    ev = ev * (i < nvec_valid).astype(jnp.float32)
                ev_v[r, pl.ds(c * L, L)] = ev
                return 0
            lax.fori_loop(0, CH * 8, vec, 0)

        def scatter_rows():
            descs = [
                pltpu.async_copy(ev_v.at[r], acc_s.at[fun_v.at[r]], sem,
                                 add=True)
                for r in range(CH)
            ]
            for d in descs:
                d.wait()

        def chunk(g, _):
            row0 = (ch0 + g) * CH
            pltpu.sync_copy(gm_h.at[0, pl.ds(row0, CH)], var_v)
            pltpu.sync_copy(gm_h.at[1, pl.ds(row0, CH)], fun_v)
            pltpu.sync_copy(ef_h.at[pl.ds(row0, CH)], ef_v)
            compute_vectors(None)
            scatter_rows()
            return 0
        lax.fori_loop(0, nch, chunk, 0)

        if tail_rows:
            @pl.when(wid == NW - 1)
            def _():
                row0 = full_chunks * CH
                pltpu.sync_copy(gm_h.at[0, pl.ds(row0, tail_rows)],
                                var_v.at[pl.ds(0, tail_rows)])
                pltpu.sync_copy(gm_h.at[1, pl.ds(row0, tail_rows)],
                                fun_v.at[pl.ds(0, tail_rows)])
                pltpu.sync_copy(ef_h.at[pl.ds(row0, tail_rows)],
                                ef_v.at[pl.ds(0, tail_rows)])
                compute_vectors(tail_rows * 8)
                scatter_rows()

        plsc.subcore_barrier()

        for c, out_h in ((0, out0_h), (1, out1_h)):
            @pl.when(cid == c)
            def _(out_h=out_h):
                def wb(j, _):
                    off = base + j * 2048
                    pltpu.sync_copy(acc_s.at[pl.ds(off, 2048)], zb_v)
                    pltpu.sync_copy(zb_v, out_h.at[pl.ds(off, 2048)])
                    return 0
                lax.fori_loop(0, nfull_z, wb, 0)
                if z_tail:
                    @pl.when(sid != NS - 1)
                    def _():
                        off = base + z_full * 2048
                        pltpu.sync_copy(acc_s.at[pl.ds(off, z_tail)],
                                        zb_v.at[pl.ds(0, z_tail)])
                        pltpu.sync_copy(zb_v.at[pl.ds(0, z_tail)],
                                        out_h.at[pl.ds(off, z_tail)])
                if zl_tail:
                    @pl.when(sid == NS - 1)
                    def _():
                        off = base + zl_full * 2048
                        pltpu.sync_copy(acc_s.at[pl.ds(off, zl_tail)],
                                        zb_v.at[pl.ds(0, zl_tail)])
                        pltpu.sync_copy(zb_v.at[pl.ds(0, zl_tail)],
                                        out_h.at[pl.ds(off, zl_tail)])

    return k


@functools.lru_cache(maxsize=None)
def _finalize(V, F):
    span = ((F + NS - 1) // NS + 7) // 8 * 8
    last = F - (NS - 1) * span

    mesh = plsc.VectorSubcoreMesh(core_axis_name="c", subcore_axis_name="s",
                                  num_cores=1)

    @functools.partial(
        pl.kernel,
        out_type=(jax.ShapeDtypeStruct((F,), jnp.float32),
                  jax.ShapeDtypeStruct((BATCH,), jnp.float32),
                  jax.ShapeDtypeStruct((BATCH,), jnp.float32)),
        mesh=mesh,
        scratch_types=[
            pltpu.VMEM((span,), jnp.float32),
            pltpu.VMEM((span,), jnp.float32),
            pltpu.VMEM((span,), jnp.int32),
            pltpu.VMEM((span,), jnp.float32),
            pltpu.VMEM((2, BATCH, L), jnp.float32),
            pltpu.VMEM((2 * BATCH,), jnp.float32),
            pltpu.VMEM((NS * 2 * BATCH,), jnp.float32),
            pltpu.VMEM((L,), jnp.int32),
            pltpu.VMEM_SHARED((NS * 2 * BATCH,), jnp.float32),
        ],
        compiler_params=pltpu.CompilerParams(needs_layout_passes=False),
    )
    def k(part0_h, part1_h, bfm_h, bvm_h, cv_h, sat_h, dif_h,
          p0_v, p1_v, bf_v, cv_v, acc_v, st_v, red_v, bv_v, sh_s):
        sid = lax.axis_index("s")
        base = sid * span

        @pl.when(sid != NS - 1)
        def _():
            pltpu.sync_copy(part0_h.at[pl.ds(base, span)], p0_v)
            pltpu.sync_copy(part1_h.at[pl.ds(base, span)], p1_v)
            pltpu.sync_copy(bfm_h.at[pl.ds(base, span)], bf_v)

        @pl.when(sid == NS - 1)
        def _():
            pltpu.sync_copy(part0_h.at[pl.ds(base, last)],
                            p0_v.at[pl.ds(0, last)])
            pltpu.sync_copy(part1_h.at[pl.ds(base, last)],
                            p1_v.at[pl.ds(0, last)])
            pltpu.sync_copy(bfm_h.at[pl.ds(base, last)],
                            bf_v.at[pl.ds(0, last)])

        for a in range(2):
            for b in range(BATCH):
                acc_v[a, b, pl.ds(0, L)] = jnp.zeros((L,), jnp.float32)

        nvec = jnp.where(sid == NS - 1, last // L, span // L)

        def vec(i, _):
            s = p0_v[pl.ds(i * L, L)] + p1_v[pl.ds(i * L, L)]
            cv = jnp.where(s > 0.0, 1.0, 0.0)
            cv_v[pl.ds(i * L, L)] = cv
            bf = bf_v[pl.ds(i * L, L)]
            for b in range(BATCH):
                mf = jnp.where(bf == b, 1.0, 0.0)
                plsc.addupdate(acc_v.at[0, b], mf * cv)
                plsc.addupdate(acc_v.at[1, b], mf)
            return 0
        lax.fori_loop(0, nvec, vec, 0)

        lanes = lax.iota(jnp.int32, L)
        for a in range(2):
            packed = jnp.zeros((L,), jnp.float32)
            for b in range(BATCH):
                s = jnp.sum(acc_v[a, b, pl.ds(0, L)])
                packed = jnp.where(lanes == b, s, packed)
            st_v[pl.ds(a * BATCH, BATCH)] = packed

        pltpu.sync_copy(st_v, sh_s.at[pl.ds(sid * 2 * BATCH, 2 * BATCH)])
        plsc.subcore_barrier()

        @pl.when(sid != NS - 1)
        def _():
            pltpu.sync_copy(cv_v, cv_h.at[pl.ds(base, span)])

        @pl.when(sid == NS - 1)
        def _():
            pltpu.sync_copy(cv_v.at[pl.ds(0, last)], cv_h.at[pl.ds(base, last)])

        @pl.when(sid == 0)
        def _():
            pltpu.sync_copy(sh_s, red_v)
            sums = jnp.zeros((L,), jnp.float32)
            cnts = jnp.zeros((L,), jnp.float32)
            for t in range(NS):
                sums = sums + red_v[pl.ds(t * 2 * BATCH, BATCH)]
                cnts = cnts + red_v[pl.ds(t * 2 * BATCH + BATCH, BATCH)]
            pltpu.sync_copy(bvm_h.at[pl.ds(V - L, L)], bv_v)
            mx = jnp.max(bv_v[pl.ds(0, L)])
            bs = ((mx + 1) // BATCH).astype(jnp.float32)
            max_sat = bs * cnts
            sat = jnp.where(max_sat == sums, 1.0, 0.0)
            dif = max_sat - sums
            st_v[pl.ds(0, BATCH)] = sat
            st_v[pl.ds(BATCH, BATCH)] = dif
            pltpu.sync_copy(st_v.at[pl.ds(0, BATCH)], sat_h)
            pltpu.sync_copy(st_v.at[pl.ds(BATCH, BATCH)], dif_h)

    return k


def kernel(variable_prediction, graph_map, batch_variable_map,
           batch_function_map, edge_feature):
    V = variable_prediction.shape[0]
    E = graph_map.shape[1]
    F = batch_function_map.shape[0]

    vp = variable_prediction[:, 0]
    gm3 = graph_map.reshape(2, E // 128, 128)
    ef2 = edge_feature[:, 0].reshape(E // 128, 128)

    partial0, partial1 = _edge_pass(V, F, E)(vp, gm3, ef2)
    cv, sat, dif = _finalize(V, F)(partial0, partial1, batch_function_map,
                                   batch_variable_map)
    return ((sat[:, None], dif[:, None], graph_map, cv[:, None]), None)


# speedup vs baseline: 224.6192x; 1.7349x over previous
import functools

import jax
import jax.numpy as jnp
from jax import lax
from jax.experimental import pallas as pl
from jax.experimental.pallas import tpu as pltpu
from jax.experimental.pallas import tpu_sc as plsc

NC = 2
NS = 16
L = 16
BATCH = 16


@functools.lru_cache(maxsize=None)
def _edge_pass(V, F, E):
    R = E // 128
    CH = 8
    NW = NC * NS
    full_chunks = R // CH
    tail_rows = R % CH
    per = full_chunks // NW
    rem = full_chunks % NW

    span = ((F + NS - 1) // NS + 7) // 8 * 8
    last = F - (NS - 1) * span
    z_full, z_tail = span // 2048, span % 2048
    zl_full, zl_tail = last // 2048, last % 2048

    mesh = plsc.VectorSubcoreMesh(core_axis_name="c", subcore_axis_name="s")

    NBUF = 3

    @functools.partial(
        pl.kernel,
        out_type=(jax.ShapeDtypeStruct((F,), jnp.float32),
                  jax.ShapeDtypeStruct((F,), jnp.float32)),
        mesh=mesh,
        scratch_types=(
            [pltpu.VMEM((V,), jnp.float32)]
            + [pltpu.VMEM((CH, 128), jnp.int32)] * NBUF
            + [pltpu.VMEM((CH, 128), jnp.int32)] * NBUF
            + [pltpu.VMEM((CH, 128), jnp.float32)] * NBUF
            + [pltpu.VMEM((CH, 128), jnp.float32)] * NBUF
            + [pltpu.VMEM((2048,), jnp.float32)]
            + [pltpu.VMEM_SHARED((F,), jnp.float32)]
            + [pltpu.SemaphoreType.DMA] * NBUF
            + [pltpu.SemaphoreType.DMA] * NBUF
            + [pltpu.SemaphoreType.DMA]
        ),
        compiler_params=pltpu.CompilerParams(needs_layout_passes=False),
    )
    def k(vp_h, gm_h, ef_h, out0_h, out1_h, vp_v,
          var0, var1, var2, fun0, fun1, fun2, ef0, ef1, ef2, ev0, ev1, ev2,
          zb_v, acc_s, isem0, isem1, isem2, ssem0, ssem1, ssem2, vp_sem):
        var_b = (var0, var1, var2)
        fun_b = (fun0, fun1, fun2)
        ef_b = (ef0, ef1, ef2)
        ev_b = (ev0, ev1, ev2)
        isem = (isem0, isem1, isem2)
        ssem = (ssem0, ssem1, ssem2)

        cid = lax.axis_index("c")
        sid = lax.axis_index("s")
        wid = sid * NC + cid

        nch = per + jnp.where(wid < rem, 1, 0)
        ch0 = wid * per + jnp.minimum(wid, rem)

        def issue_inputs(g, b):
            row0 = (ch0 + g) * CH
            pltpu.async_copy(gm_h.at[0, pl.ds(row0, CH)], var_b[b], isem[b])
            pltpu.async_copy(gm_h.at[1, pl.ds(row0, CH)], fun_b[b], isem[b])
            pltpu.async_copy(ef_h.at[pl.ds(row0, CH)], ef_b[b], isem[b])

        def drain_inputs(b):
            pltpu.make_async_copy(gm_h.at[0, pl.ds(0, CH)], var_b[b],
                                  isem[b]).wait()
            pltpu.make_async_copy(gm_h.at[1, pl.ds(0, CH)], fun_b[b],
                                  isem[b]).wait()
            pltpu.make_async_copy(ef_h.at[pl.ds(0, CH)], ef_b[b],
                                  isem[b]).wait()

        def drain_scatter(b):
            pltpu.make_async_copy(ef_h.at[pl.ds(0, CH)], ev_b[b],
                                  ssem[b]).wait()

        issue_inputs(0, 0)
        vp_d = pltpu.async_copy(vp_h, vp_v, vp_sem)

        def zb_body(i, _):
            zb_v[pl.ds(i * L, L)] = jnp.zeros((L,), jnp.float32)
            return 0
        lax.fori_loop(0, 2048 // L, zb_body, 0)

        base = sid * span

        def z_body(j, _):
            pltpu.sync_copy(zb_v, acc_s.at[pl.ds(base + j * 2048, 2048)])
            return 0
        nfull_z = jnp.where(sid == NS - 1, zl_full, z_full)
        lax.fori_loop(0, nfull_z, z_body, 0)
        if z_tail:
            @pl.when(sid != NS - 1)
            def _():
                pltpu.sync_copy(zb_v.at[pl.ds(0, z_tail)],
                                acc_s.at[pl.ds(base + z_full * 2048, z_tail)])
        if zl_tail:
            @pl.when(sid == NS - 1)
            def _():
                pltpu.sync_copy(zb_v.at[pl.ds(0, zl_tail)],
                                acc_s.at[pl.ds(base + zl_full * 2048, zl_tail)])

        vp_d.wait()
        plsc.subcore_barrier()

        def compute_vectors(b, nvec_valid):
            def vec(i, _):
                r = i // 8
                c = i % 8
                idx = var_b[b][r, pl.ds(c * L, L)]
                vv = plsc.load_gather(vp_v, [idx])
                e = ef_b[b][r, pl.ds(c * L, L)]
                t = e * vv + (1.0 - e) * 0.5
                ev = jnp.where(t > 0.5, 1.0, 0.0)
                if nvec_valid is not None:
                    ev = ev * (i < nvec_valid).astype(jnp.float32)
                ev_b[b][r, pl.ds(c * L, L)] = ev
                return 0
            lax.fori_loop(0, CH * 8, vec, 0)

        def scatter_rows(b):
            for r in range(CH):
                pltpu.async_copy(ev_b[b].at[r], acc_s.at[fun_b[b].at[r]],
                                 ssem[b], add=True)

        def chunk(g, _):
            for X in range(NBUF):
                @pl.when(g % NBUF == X)
                def _(X=X):
                    bn = (X + 1) % NBUF

                    @pl.when(g >= 2)
                    def _():
                        drain_scatter(bn)

                    @pl.when(g + 1 < nch)
                    def _():
                        issue_inputs(g + 1, bn)

                    drain_inputs(X)
                    compute_vectors(X, None)
                    scatter_rows(X)
            return 0
        lax.fori_loop(0, nch, chunk, 0)

        if rem > 0:
            @pl.when(wid < rem)
            def _():
                drain_scatter((per - 1) % NBUF)
                drain_scatter(per % NBUF)

        @pl.when(wid >= rem)
        def _():
            drain_scatter((per - 2) % NBUF)
            drain_scatter((per - 1) % NBUF)

        if tail_rows:
            @pl.when(wid == NW - 1)
            def _():
                row0 = full_chunks * CH
                pltpu.sync_copy(gm_h.at[0, pl.ds(row0, tail_rows)],
                                var_b[0].at[pl.ds(0, tail_rows)])
                pltpu.sync_copy(gm_h.at[1, pl.ds(row0, tail_rows)],
                                fun_b[0].at[pl.ds(0, tail_rows)])
                pltpu.sync_copy(ef_h.at[pl.ds(row0, tail_rows)],
                                ef_b[0].at[pl.ds(0, tail_rows)])
                compute_vectors(0, tail_rows * 8)
                scatter_rows(0)
                drain_scatter(0)

        plsc.subcore_barrier()

        for c, out_h in ((0, out0_h), (1, out1_h)):
            @pl.when(cid == c)
            def _(out_h=out_h):
                def wb(j, _):
                    off = base + j * 2048
                    pltpu.sync_copy(acc_s.at[pl.ds(off, 2048)], zb_v)
                    pltpu.sync_copy(zb_v, out_h.at[pl.ds(off, 2048)])
                    return 0
                lax.fori_loop(0, nfull_z, wb, 0)
                if z_tail:
                    @pl.when(sid != NS - 1)
                    def _():
                        off = base + z_full * 2048
                        pltpu.sync_copy(acc_s.at[pl.ds(off, z_tail)],
                                        zb_v.at[pl.ds(0, z_tail)])
                        pltpu.sync_copy(zb_v.at[pl.ds(0, z_tail)],
                                        out_h.at[pl.ds(off, z_tail)])
                if zl_tail:
                    @pl.when(sid == NS - 1)
                    def _():
                        off = base + zl_full * 2048
                        pltpu.sync_copy(acc_s.at[pl.ds(off, zl_tail)],
                                        zb_v.at[pl.ds(0, zl_tail)])
                        pltpu.sync_copy(zb_v.at[pl.ds(0, zl_tail)],
                                        out_h.at[pl.ds(off, zl_tail)])

    return k


@functools.lru_cache(maxsize=None)
def _finalize(V, F):
    span = ((F + NS - 1) // NS + 7) // 8 * 8
    last = F - (NS - 1) * span

    mesh = plsc.VectorSubcoreMesh(core_axis_name="c", subcore_axis_name="s",
                                  num_cores=1)

    @functools.partial(
        pl.kernel,
        out_type=(jax.ShapeDtypeStruct((F,), jnp.float32),
                  jax.ShapeDtypeStruct((BATCH,), jnp.float32),
                  jax.ShapeDtypeStruct((BATCH,), jnp.float32)),
        mesh=mesh,
        scratch_types=[
            pltpu.VMEM((span,), jnp.float32),
            pltpu.VMEM((span,), jnp.float32),
            pltpu.VMEM((span,), jnp.int32),
            pltpu.VMEM((span,), jnp.float32),
            pltpu.VMEM((2, BATCH, L), jnp.float32),
            pltpu.VMEM((2 * BATCH,), jnp.float32),
            pltpu.VMEM((NS * 2 * BATCH,), jnp.float32),
            pltpu.VMEM((L,), jnp.int32),
            pltpu.VMEM_SHARED((NS * 2 * BATCH,), jnp.float32),
        ],
        compiler_params=pltpu.CompilerParams(needs_layout_passes=False),
    )
    def k(part0_h, part1_h, bfm_h, bvm_h, cv_h, sat_h, dif_h,
          p0_v, p1_v, bf_v, cv_v, acc_v, st_v, red_v, bv_v, sh_s):
        sid = lax.axis_index("s")
        base = sid * span

        @pl.when(sid != NS - 1)
        def _():
            pltpu.sync_copy(part0_h.at[pl.ds(base, span)], p0_v)
            pltpu.sync_copy(part1_h.at[pl.ds(base, span)], p1_v)
            pltpu.sync_copy(bfm_h.at[pl.ds(base, span)], bf_v)

        @pl.when(sid == NS - 1)
        def _():
            pltpu.sync_copy(part0_h.at[pl.ds(base, last)],
                            p0_v.at[pl.ds(0, last)])
            pltpu.sync_copy(part1_h.at[pl.ds(base, last)],
                            p1_v.at[pl.ds(0, last)])
            pltpu.sync_copy(bfm_h.at[pl.ds(base, last)],
                            bf_v.at[pl.ds(0, last)])

        for a in range(2):
            for b in range(BATCH):
                acc_v[a, b, pl.ds(0, L)] = jnp.zeros((L,), jnp.float32)

        nvec = jnp.where(sid == NS - 1, last // L, span // L)

        def vec(i, _):
            s = p0_v[pl.ds(i * L, L)] + p1_v[pl.ds(i * L, L)]
            cv = jnp.where(s > 0.0, 1.0, 0.0)
            cv_v[pl.ds(i * L, L)] = cv
            bf = bf_v[pl.ds(i * L, L)]
            for b in range(BATCH):
                mf = jnp.where(bf == b, 1.0, 0.0)
                plsc.addupdate(acc_v.at[0, b], mf * cv)
                plsc.addupdate(acc_v.at[1, b], mf)
            return 0
        lax.fori_loop(0, nvec, vec, 0)

        lanes = lax.iota(jnp.int32, L)
        for a in range(2):
            packed = jnp.zeros((L,), jnp.float32)
            for b in range(BATCH):
                s = jnp.sum(acc_v[a, b, pl.ds(0, L)])
                packed = jnp.where(lanes == b, s, packed)
            st_v[pl.ds(a * BATCH, BATCH)] = packed

        pltpu.sync_copy(st_v, sh_s.at[pl.ds(sid * 2 * BATCH, 2 * BATCH)])
        plsc.subcore_barrier()

        @pl.when(sid != NS - 1)
        def _():
            pltpu.sync_copy(cv_v, cv_h.at[pl.ds(base, span)])

        @pl.when(sid == NS - 1)
        def _():
            pltpu.sync_copy(cv_v.at[pl.ds(0, last)], cv_h.at[pl.ds(base, last)])

        @pl.when(sid == 0)
        def _():
            pltpu.sync_copy(sh_s, red_v)
            sums = jnp.zeros((L,), jnp.float32)
            cnts = jnp.zeros((L,), jnp.float32)
            for t in range(NS):
                sums = sums + red_v[pl.ds(t * 2 * BATCH, BATCH)]
                cnts = cnts + red_v[pl.ds(t * 2 * BATCH + BATCH, BATCH)]
            pltpu.sync_copy(bvm_h.at[pl.ds(V - L, L)], bv_v)
            mx = jnp.max(bv_v[pl.ds(0, L)])
            bs = ((mx + 1) // BATCH).astype(jnp.float32)
            max_sat = bs * cnts
            sat = jnp.where(max_sat == sums, 1.0, 0.0)
            dif = max_sat - sums
            st_v[pl.ds(0, BATCH)] = sat
            st_v[pl.ds(BATCH, BATCH)] = dif
            pltpu.sync_copy(st_v.at[pl.ds(0, BATCH)], sat_h)
            pltpu.sync_copy(st_v.at[pl.ds(BATCH, BATCH)], dif_h)

    return k


def kernel(variable_prediction, graph_map, batch_variable_map,
           batch_function_map, edge_feature):
    V = variable_prediction.shape[0]
    E = graph_map.shape[1]
    F = batch_function_map.shape[0]

    vp = variable_prediction[:, 0]
    gm3 = graph_map.reshape(2, E // 128, 128)
    ef2 = edge_feature[:, 0].reshape(E // 128, 128)

    partial0, partial1 = _edge_pass(V, F, E)(vp, gm3, ef2)
    cv, sat, dif = _finalize(V, F)(partial0, partial1, batch_function_map,
                                   batch_variable_map)
    return ((sat[:, None], dif[:, None], graph_map, cv[:, None]), None)


# speedup vs baseline: 234.5371x; 1.0442x over previous
import functools

import jax
import jax.numpy as jnp
from jax import lax
from jax.experimental import pallas as pl
from jax.experimental.pallas import tpu as pltpu
from jax.experimental.pallas import tpu_sc as plsc

NC = 2
NS = 16
L = 16
BATCH = 16


@functools.lru_cache(maxsize=None)
def _edge_pass(V, F, E):
    R = E // 128
    CH = 8
    NW = NC * NS
    full_chunks = R // CH
    tail_rows = R % CH
    per = full_chunks // NW
    rem = full_chunks % NW

    span = ((F + NS - 1) // NS + 7) // 8 * 8
    last = F - (NS - 1) * span
    z_full, z_tail = span // 2048, span % 2048
    zl_full, zl_tail = last // 2048, last % 2048

    mesh = plsc.VectorSubcoreMesh(core_axis_name="c", subcore_axis_name="s")

    NBUF = 3

    @functools.partial(
        pl.kernel,
        out_type=(jax.ShapeDtypeStruct((F,), jnp.float32),
                  jax.ShapeDtypeStruct((F,), jnp.float32)),
        mesh=mesh,
        scratch_types=(
            [pltpu.VMEM((V,), jnp.float32)]
            + [pltpu.VMEM((2, CH * 128), jnp.int32)] * NBUF
            + [pltpu.VMEM((CH * 128,), jnp.float32)] * NBUF
            + [pltpu.VMEM((CH * 128,), jnp.float32)] * NBUF
            + [pltpu.VMEM((2048,), jnp.float32)]
            + [pltpu.VMEM_SHARED((F,), jnp.float32)]
            + [pltpu.SemaphoreType.DMA] * NBUF
            + [pltpu.SemaphoreType.DMA] * NBUF
            + [pltpu.SemaphoreType.DMA]
        ),
        compiler_params=pltpu.CompilerParams(needs_layout_passes=False),
    )
    def k(vp_h, gm_h, ef_h, out0_h, out1_h, vp_v,
          gmb0, gmb1, gmb2, ef0, ef1, ef2, ev0, ev1, ev2,
          zb_v, acc_s, isem0, isem1, isem2, ssem0, ssem1, ssem2, vp_sem):
        gm_b = (gmb0, gmb1, gmb2)
        ef_b = (ef0, ef1, ef2)
        ev_b = (ev0, ev1, ev2)
        isem = (isem0, isem1, isem2)
        ssem = (ssem0, ssem1, ssem2)

        cid = lax.axis_index("c")
        sid = lax.axis_index("s")
        wid = sid * NC + cid

        nch = per + jnp.where(wid < rem, 1, 0)
        ch0 = wid * per + jnp.minimum(wid, rem)

        CW = CH * 128

        def issue_inputs(g, b):
            e0 = (ch0 + g) * CW
            pltpu.async_copy(gm_h.at[:, pl.ds(e0, CW)], gm_b[b], isem[b])
            pltpu.async_copy(ef_h.at[pl.ds(e0, CW)], ef_b[b], isem[b])

        def drain_inputs(b):
            pltpu.make_async_copy(gm_h.at[:, pl.ds(0, CW)], gm_b[b],
                                  isem[b]).wait()
            pltpu.make_async_copy(ef_h.at[pl.ds(0, CW)], ef_b[b],
                                  isem[b]).wait()

        def drain_scatter(b):
            pltpu.make_async_copy(ef_h.at[pl.ds(0, CW)], ev_b[b],
                                  ssem[b]).wait()

        issue_inputs(0, 0)
        vp_d = pltpu.async_copy(vp_h, vp_v, vp_sem)

        def zb_body(i, _):
            zb_v[pl.ds(i * L, L)] = jnp.zeros((L,), jnp.float32)
            return 0
        lax.fori_loop(0, 2048 // L, zb_body, 0)

        base = sid * span

        def z_body(j, _):
            pltpu.sync_copy(zb_v, acc_s.at[pl.ds(base + j * 2048, 2048)])
            return 0
        nfull_z = jnp.where(sid == NS - 1, zl_full, z_full)
        lax.fori_loop(0, nfull_z, z_body, 0)
        if z_tail:
            @pl.when(sid != NS - 1)
            def _():
                pltpu.sync_copy(zb_v.at[pl.ds(0, z_tail)],
                                acc_s.at[pl.ds(base + z_full * 2048, z_tail)])
        if zl_tail:
            @pl.when(sid == NS - 1)
            def _():
                pltpu.sync_copy(zb_v.at[pl.ds(0, zl_tail)],
                                acc_s.at[pl.ds(base + zl_full * 2048, zl_tail)])

        vp_d.wait()
        plsc.subcore_barrier()

        def compute_vectors(b, nvec_valid):
            def vec(i, _):
                idx = gm_b[b][0, pl.ds(i * L, L)]
                vv = plsc.load_gather(vp_v, [idx])
                e = ef_b[b][pl.ds(i * L, L)]
                t = e * vv + (1.0 - e) * 0.5
                ev = jnp.where(t > 0.5, 1.0, 0.0)
                if nvec_valid is not None:
                    ev = ev * (i < nvec_valid).astype(jnp.float32)
                ev_b[b][pl.ds(i * L, L)] = ev
                return 0
            lax.fori_loop(0, CH * 8, vec, 0)

        def scatter_rows(b):
            for r in range(CH):
                pltpu.async_copy(
                    ev_b[b].at[pl.ds(r * 128, 128)],
                    acc_s.at[gm_b[b].at[1, pl.ds(r * 128, 128)]],
                    ssem[b], add=True)

        def chunk(g, _):
            for X in range(NBUF):
                @pl.when(g % NBUF == X)
                def _(X=X):
                    bn = (X + 1) % NBUF

                    @pl.when(g >= 2)
                    def _():
                        drain_scatter(bn)

                    @pl.when(g + 1 < nch)
                    def _():
                        issue_inputs(g + 1, bn)

                    drain_inputs(X)
                    compute_vectors(X, None)
                    scatter_rows(X)
            return 0
        lax.fori_loop(0, nch, chunk, 0)

        if rem > 0:
            @pl.when(wid < rem)
            def _():
                drain_scatter((per - 1) % NBUF)
                drain_scatter(per % NBUF)

        @pl.when(wid >= rem)
        def _():
            drain_scatter((per - 2) % NBUF)
            drain_scatter((per - 1) % NBUF)

        if tail_rows:
            @pl.when(wid == NW - 1)
            def _():
                e0 = full_chunks * CW
                tw = tail_rows * 128
                pltpu.sync_copy(gm_h.at[:, pl.ds(e0, tw)],
                                gm_b[0].at[:, pl.ds(0, tw)])
                pltpu.sync_copy(ef_h.at[pl.ds(e0, tw)],
                                ef_b[0].at[pl.ds(0, tw)])
                compute_vectors(0, tail_rows * 8)
                scatter_rows(0)
                drain_scatter(0)

        plsc.subcore_barrier()

        for c, out_h in ((0, out0_h), (1, out1_h)):
            @pl.when(cid == c)
            def _(out_h=out_h):
                def wb(j, _):
                    off = base + j * 2048
                    pltpu.sync_copy(acc_s.at[pl.ds(off, 2048)], zb_v)
                    pltpu.sync_copy(zb_v, out_h.at[pl.ds(off, 2048)])
                    return 0
                lax.fori_loop(0, nfull_z, wb, 0)
                if z_tail:
                    @pl.when(sid != NS - 1)
                    def _():
                        off = base + z_full * 2048
                        pltpu.sync_copy(acc_s.at[pl.ds(off, z_tail)],
                                        zb_v.at[pl.ds(0, z_tail)])
                        pltpu.sync_copy(zb_v.at[pl.ds(0, z_tail)],
                                        out_h.at[pl.ds(off, z_tail)])
                if zl_tail:
                    @pl.when(sid == NS - 1)
                    def _():
                        off = base + zl_full * 2048
                        pltpu.sync_copy(acc_s.at[pl.ds(off, zl_tail)],
                                        zb_v.at[pl.ds(0, zl_tail)])
                        pltpu.sync_copy(zb_v.at[pl.ds(0, zl_tail)],
                                        out_h.at[pl.ds(off, zl_tail)])

    return k


@functools.lru_cache(maxsize=None)
def _finalize(V, F):
    span = ((F + NS - 1) // NS + 7) // 8 * 8
    last = F - (NS - 1) * span

    mesh = plsc.VectorSubcoreMesh(core_axis_name="c", subcore_axis_name="s",
                                  num_cores=1)

    @functools.partial(
        pl.kernel,
        out_type=(jax.ShapeDtypeStruct((F,), jnp.float32),
                  jax.ShapeDtypeStruct((BATCH,), jnp.float32),
                  jax.ShapeDtypeStruct((BATCH,), jnp.float32)),
        mesh=mesh,
        scratch_types=[
            pltpu.VMEM((span,), jnp.float32),
            pltpu.VMEM((span,), jnp.float32),
            pltpu.VMEM((span,), jnp.int32),
            pltpu.VMEM((span,), jnp.float32),
            pltpu.VMEM((2, BATCH, L), jnp.float32),
            pltpu.VMEM((2 * BATCH,), jnp.float32),
            pltpu.VMEM((NS * 2 * BATCH,), jnp.float32),
            pltpu.VMEM((L,), jnp.int32),
            pltpu.VMEM_SHARED((NS * 2 * BATCH,), jnp.float32),
        ],
        compiler_params=pltpu.CompilerParams(needs_layout_passes=False),
    )
    def k(part0_h, part1_h, bfm_h, bvm_h, cv_h, sat_h, dif_h,
          p0_v, p1_v, bf_v, cv_v, acc_v, st_v, red_v, bv_v, sh_s):
        sid = lax.axis_index("s")
        base = sid * span

        @pl.when(sid != NS - 1)
        def _():
            pltpu.sync_copy(part0_h.at[pl.ds(base, span)], p0_v)
            pltpu.sync_copy(part1_h.at[pl.ds(base, span)], p1_v)
            pltpu.sync_copy(bfm_h.at[pl.ds(base, span)], bf_v)

        @pl.when(sid == NS - 1)
        def _():
            pltpu.sync_copy(part0_h.at[pl.ds(base, last)],
                            p0_v.at[pl.ds(0, last)])
            pltpu.sync_copy(part1_h.at[pl.ds(base, last)],
                            p1_v.at[pl.ds(0, last)])
            pltpu.sync_copy(bfm_h.at[pl.ds(base, last)],
                            bf_v.at[pl.ds(0, last)])

        for a in range(2):
            for b in range(BATCH):
                acc_v[a, b, pl.ds(0, L)] = jnp.zeros((L,), jnp.float32)

        nvec = jnp.where(sid == NS - 1, last // L, span // L)

        def vec(i, _):
            s = p0_v[pl.ds(i * L, L)] + p1_v[pl.ds(i * L, L)]
            cv = jnp.where(s > 0.0, 1.0, 0.0)
            cv_v[pl.ds(i * L, L)] = cv
            bf = bf_v[pl.ds(i * L, L)]
            for b in range(BATCH):
                mf = jnp.where(bf == b, 1.0, 0.0)
                plsc.addupdate(acc_v.at[0, b], mf * cv)
                plsc.addupdate(acc_v.at[1, b], mf)
            return 0
        lax.fori_loop(0, nvec, vec, 0)

        lanes = lax.iota(jnp.int32, L)
        for a in range(2):
            packed = jnp.zeros((L,), jnp.float32)
            for b in range(BATCH):
                s = jnp.sum(acc_v[a, b, pl.ds(0, L)])
                packed = jnp.where(lanes == b, s, packed)
            st_v[pl.ds(a * BATCH, BATCH)] = packed

        pltpu.sync_copy(st_v, sh_s.at[pl.ds(sid * 2 * BATCH, 2 * BATCH)])
        plsc.subcore_barrier()

        @pl.when(sid != NS - 1)
        def _():
            pltpu.sync_copy(cv_v, cv_h.at[pl.ds(base, span)])

        @pl.when(sid == NS - 1)
        def _():
            pltpu.sync_copy(cv_v.at[pl.ds(0, last)], cv_h.at[pl.ds(base, last)])

        @pl.when(sid == 0)
        def _():
            pltpu.sync_copy(sh_s, red_v)
            sums = jnp.zeros((L,), jnp.float32)
            cnts = jnp.zeros((L,), jnp.float32)
            for t in range(NS):
                sums = sums + red_v[pl.ds(t * 2 * BATCH, BATCH)]
                cnts = cnts + red_v[pl.ds(t * 2 * BATCH + BATCH, BATCH)]
            pltpu.sync_copy(bvm_h.at[pl.ds(V - L, L)], bv_v)
            mx = jnp.max(bv_v[pl.ds(0, L)])
            bs = ((mx + 1) // BATCH).astype(jnp.float32)
            max_sat = bs * cnts
            sat = jnp.where(max_sat == sums, 1.0, 0.0)
            dif = max_sat - sums
            st_v[pl.ds(0, BATCH)] = sat
            st_v[pl.ds(BATCH, BATCH)] = dif
            pltpu.sync_copy(st_v.at[pl.ds(0, BATCH)], sat_h)
            pltpu.sync_copy(st_v.at[pl.ds(BATCH, BATCH)], dif_h)

    return k


def kernel(variable_prediction, graph_map, batch_variable_map,
           batch_function_map, edge_feature):
    V = variable_prediction.shape[0]
    E = graph_map.shape[1]
    F = batch_function_map.shape[0]

    vp = variable_prediction[:, 0]
    ef1 = edge_feature[:, 0]

    partial0, partial1 = _edge_pass(V, F, E)(vp, graph_map, ef1)
    cv, sat, dif = _finalize(V, F)(partial0, partial1, batch_function_map,
                                   batch_variable_map)
    return ((sat[:, None], dif[:, None], graph_map, cv[:, None]), None)
